# Initial kernel scaffold; baseline (speedup 1.0000x reference)
import jax
import jax.numpy as jnp
from jax.experimental import pallas as pl


def kernel(component_labels, emb_weight, norm_weight):
    raise NotImplementedError("write your pallas kernel here")


# speedup vs baseline: 6.0868x; 6.0868x over previous
import functools

import jax
import jax.numpy as jnp
from jax import lax
from jax.experimental import pallas as pl
from jax.experimental.pallas import tpu as pltpu
from jax.experimental.pallas import tpu_sc as plsc

_EPS = float(jnp.finfo(jnp.float32).eps)

_N_ROWS = 100000
_DIM = 32


def _norm_body(tab_ref, nw_ref, out_ref):
    x = tab_ref[...]
    ms = jnp.mean(x * x, axis=-1, keepdims=True)
    out_ref[...] = x * lax.rsqrt(ms + _EPS) * nw_ref[...]


def _normalize_table(emb_weight, norm_weight):
    blk = 10000
    grid = _N_ROWS // blk
    return pl.pallas_call(
        _norm_body,
        grid=(grid,),
        in_specs=[
            pl.BlockSpec((blk, _DIM), lambda i: (i, 0)),
            pl.BlockSpec((1, _DIM), lambda i: (0, 0)),
        ],
        out_specs=pl.BlockSpec((blk, _DIM), lambda i: (i, 0)),
        out_shape=jax.ShapeDtypeStruct((_N_ROWS, _DIM), jnp.float32),
    )(emb_weight, norm_weight.reshape(1, _DIM))


def _make_gather(b_total):
    info = plsc.get_sparse_core_info()
    nc, ns = info.num_cores, info.num_subcores
    nw = nc * ns
    irows_total = b_total // 128
    irows_per_w = irows_total // nw
    k = 8
    steps = irows_per_w // k
    chunk = k * 128

    mesh = plsc.VectorSubcoreMesh(core_axis_name="c", subcore_axis_name="s")

    @functools.partial(
        pl.kernel,
        mesh=mesh,
        compiler_params=pltpu.CompilerParams(use_tc_tiling_on_sc=False),
        out_type=jax.ShapeDtypeStruct((b_total, _DIM), jnp.float32),
        scratch_types=[
            pltpu.VMEM((k, 128), jnp.int32),
            pltpu.VMEM((chunk, _DIM), jnp.float32),
            pltpu.SemaphoreType.DMA,
            pltpu.SemaphoreType.DMA,
        ],
    )
    def gather(tab_hbm, idx_hbm, out_hbm, idx_v, rows_v, isem, gsem):
        wid = lax.axis_index("s") * nc + lax.axis_index("c")
        irow0 = wid * irows_per_w

        def body(i, carry):
            r = irow0 + i * k
            pltpu.async_copy(idx_hbm.at[pl.ds(r, k)], idx_v, isem).wait()
            cps = [
                pltpu.async_copy(
                    tab_hbm.at[idx_v.at[j]],
                    rows_v.at[pl.ds(j * 128, 128)],
                    gsem,
                )
                for j in range(k)
            ]
            for c in cps:
                c.wait()
            pltpu.sync_copy(rows_v, out_hbm.at[pl.ds(r * 128, chunk)])
            return carry

        lax.fori_loop(0, steps, body, 0)

    return gather


def kernel(component_labels, emb_weight, norm_weight):
    b, h = component_labels.shape
    b_total = b * h
    idx = component_labels.reshape(b_total // 128, 128).astype(jnp.int32)
    tab = _normalize_table(emb_weight, norm_weight)
    out = _make_gather(b_total)(tab, idx)
    return out.reshape(b, h, _DIM)


# speedup vs baseline: 6.4102x; 1.0531x over previous
import functools

import jax
import jax.numpy as jnp
from jax import lax
from jax.experimental import pallas as pl
from jax.experimental.pallas import tpu as pltpu
from jax.experimental.pallas import tpu_sc as plsc

_EPS = float(jnp.finfo(jnp.float32).eps)

_N_ROWS = 100000
_DIM = 32


def _norm_body(tab_ref, nw_ref, out_ref):
    x = tab_ref[...]
    ms = jnp.mean(x * x, axis=-1, keepdims=True)
    out_ref[...] = x * lax.rsqrt(ms + _EPS) * nw_ref[...]


def _normalize_table(emb_weight, norm_weight):
    blk = 10000
    grid = _N_ROWS // blk
    return pl.pallas_call(
        _norm_body,
        grid=(grid,),
        in_specs=[
            pl.BlockSpec((blk, _DIM), lambda i: (i, 0)),
            pl.BlockSpec((1, _DIM), lambda i: (0, 0)),
        ],
        out_specs=pl.BlockSpec((blk, _DIM), lambda i: (i, 0)),
        out_shape=jax.ShapeDtypeStruct((_N_ROWS, _DIM), jnp.float32),
    )(emb_weight, norm_weight.reshape(1, _DIM))


def _make_gather(b_total):
    info = plsc.get_sparse_core_info()
    nc, ns = info.num_cores, info.num_subcores
    nw = nc * ns
    irows_total = b_total // 128
    irows_per_w = irows_total // nw
    k = 8
    steps = irows_per_w // k
    chunk = k * 128

    assert steps % 2 == 0 and steps >= 4
    mesh = plsc.VectorSubcoreMesh(core_axis_name="c", subcore_axis_name="s")

    @functools.partial(
        pl.kernel,
        mesh=mesh,
        compiler_params=pltpu.CompilerParams(use_tc_tiling_on_sc=False),
        out_type=jax.ShapeDtypeStruct((b_total, _DIM), jnp.float32),
        scratch_types=[
            pltpu.VMEM((2, k, 128), jnp.int32),
            pltpu.VMEM((2, chunk, _DIM), jnp.float32),
            pltpu.SemaphoreType.DMA,
            pltpu.SemaphoreType.DMA,
            pltpu.SemaphoreType.DMA,
            pltpu.SemaphoreType.DMA,
            pltpu.SemaphoreType.DMA,
        ],
    )
    def gather(tab_hbm, idx_hbm, out_hbm, idx_v, rows_v, is0, is1, gsem, os0, os1):
        wid = lax.axis_index("s") * nc + lax.axis_index("c")
        irow0 = wid * irows_per_w
        isem = (is0, is1)
        osem = (os0, os1)

        def icp(i, s):
            return pltpu.async_copy(
                idx_hbm.at[pl.ds(irow0 + i * k, k)], idx_v.at[s], isem[s]
            )

        def ocp(i, s):
            return pltpu.async_copy(
                rows_v.at[s], out_hbm.at[pl.ds((irow0 + i * k) * 128, chunk)], osem[s]
            )

        def run_gathers(s):
            cps = [
                pltpu.async_copy(
                    tab_hbm.at[idx_v.at[s, j]],
                    rows_v.at[s, pl.ds(j * 128, 128)],
                    gsem,
                )
                for j in range(k)
            ]
            for c in cps:
                c.wait()

        c0 = icp(0, 0)
        c1 = icp(1, 1)
        c0.wait()
        run_gathers(0)
        icp(2, 0)
        ocp(0, 0)
        c1.wait()
        run_gathers(1)
        icp(3, 1)
        ocp(1, 1)

        def body(g, carry):
            i = 2 * g + 2
            for s in (0, 1):
                ii = i + s
                pltpu.make_async_copy(
                    rows_v.at[s], out_hbm.at[pl.ds(0, chunk)], osem[s]
                ).wait()
                pltpu.make_async_copy(
                    idx_hbm.at[pl.ds(0, k)], idx_v.at[s], isem[s]
                ).wait()
                run_gathers(s)

                @pl.when(ii + 2 < steps)
                def _():
                    icp(ii + 2, s)

                ocp(ii, s)
            return carry

        lax.fori_loop(0, (steps - 2) // 2, body, 0)
        for s in (0, 1):
            pltpu.make_async_copy(
                rows_v.at[s], out_hbm.at[pl.ds(0, chunk)], osem[s]
            ).wait()

    return gather


def kernel(component_labels, emb_weight, norm_weight):
    b, h = component_labels.shape
    b_total = b * h
    idx = component_labels.reshape(b_total // 128, 128).astype(jnp.int32)
    tab = _normalize_table(emb_weight, norm_weight)
    out = _make_gather(b_total)(tab, idx)
    return out.reshape(b, h, _DIM)
